# Initial kernel scaffold; baseline (speedup 1.0000x reference)
import jax
import jax.numpy as jnp
from jax.experimental import pallas as pl


def kernel(z_e, embedding_weight):
    raise NotImplementedError("write your pallas kernel here")


# speedup vs baseline: 1.3670x; 1.3670x over previous
import functools

import jax
import jax.numpy as jnp
from jax import lax
from jax.experimental import pallas as pl
from jax.experimental.pallas import tpu as pltpu
from jax.experimental.pallas import tpu_sc as plsc

_N_E = 8192
_E_DIM = 256
_N_Z = 8192
_COMMITMENT_COST = 0.25

_R_BLK = 2048
_C_BLK = 1024

_T_BLK = 1024


def _argmin_body(x_ref, w_ref, zn_ref, en_ref, idx_ref, minval_ref, minidx_ref):
    j = pl.program_id(1)

    @pl.when(j == 0)
    def _():
        minval_ref[...] = jnp.full_like(minval_ref[...], jnp.inf)
        minidx_ref[...] = jnp.zeros_like(minidx_ref[...])

    xb = (x_ref[...] * -2.0).astype(jnp.bfloat16)
    wb = w_ref[...].astype(jnp.bfloat16)
    mm2 = lax.dot_general(
        xb, wb, (((1,), (1,)), ((), ())), preferred_element_type=jnp.float32
    )
    zn = zn_ref[...]
    en = en_ref[...]
    mv = minval_ref[...]
    iv = minidx_ref[...]
    lane = lax.broadcasted_iota(jnp.int32, (1, 128), 1)
    for c in range(_C_BLK // 128):
        sl = slice(c * 128, (c + 1) * 128)
        d_c = (zn + en[:, sl]) + mm2[:, sl]
        colidx = lane + (j * _C_BLK + c * 128)
        better = d_c < mv
        mv = jnp.where(better, d_c, mv)
        iv = jnp.where(better, colidx, iv)
    minval_ref[...] = mv
    minidx_ref[...] = iv

    @pl.when(j == pl.num_programs(1) - 1)
    def _():
        m = jnp.min(mv, axis=1, keepdims=True)
        li = jnp.min(
            jnp.where(mv == m, iv, jnp.int32(2**30)), axis=1, keepdims=True
        )
        idx_ref[...] = li


def _compute_indices(flatten, w, zn, en):
    grid = (_N_Z // _R_BLK, _N_E // _C_BLK)
    return pl.pallas_call(
        _argmin_body,
        grid=grid,
        in_specs=[
            pl.BlockSpec((_R_BLK, _E_DIM), lambda i, j: (i, 0)),
            pl.BlockSpec((_C_BLK, _E_DIM), lambda i, j: (j, 0)),
            pl.BlockSpec((_R_BLK, 1), lambda i, j: (i, 0)),
            pl.BlockSpec((1, _C_BLK), lambda i, j: (0, j)),
        ],
        out_specs=pl.BlockSpec((_R_BLK, 1), lambda i, j: (i, 0)),
        out_shape=jax.ShapeDtypeStruct((_N_Z, 1), jnp.int32),
        scratch_shapes=[
            pltpu.VMEM((_R_BLK, 128), jnp.float32),
            pltpu.VMEM((_R_BLK, 128), jnp.int32),
        ],
    )(flatten, w, zn, en)


_SC_MESH = plsc.VectorSubcoreMesh(core_axis_name="c", subcore_axis_name="s")
_NUM_WORKERS = 32
_B_PER_W = _N_Z // _NUM_WORKERS


def _gather_rows(w, idx):
    @functools.partial(
        pl.kernel,
        mesh=_SC_MESH,
        out_type=jax.ShapeDtypeStruct((_N_Z, _E_DIM), jnp.float32),
        scratch_types=[
            pltpu.VMEM((_B_PER_W,), jnp.int32),
            pltpu.VMEM((_B_PER_W, _E_DIM), jnp.float32),
            pltpu.SemaphoreType.DMA,
        ],
    )
    def k(table_hbm, idx_hbm, out_hbm, idx_v, rows_v, sem):
        wid = lax.axis_index("s") * 2 + lax.axis_index("c")
        base = wid * _B_PER_W
        pltpu.sync_copy(idx_hbm.at[pl.ds(base, _B_PER_W)], idx_v)
        pltpu.async_copy(table_hbm.at[idx_v], rows_v, sem).wait()
        pltpu.sync_copy(rows_v, out_hbm.at[pl.ds(base, _B_PER_W)])

    return k(w, idx)


def _loss_transpose_body(
    zq_hbm, ze_ref, zqt_ref, loss_ref, buf0, buf1, sem0, sem1
):
    i = pl.program_id(0)
    n = pl.num_programs(0)
    nxt = i + 1

    def _copy(blk, buf, sem):
        return pltpu.make_async_copy(
            zq_hbm.at[pl.ds(blk * _T_BLK, _T_BLK), :], buf, sem
        )

    @pl.when(i == 0)
    def _():
        _copy(0, buf0, sem0).start()

    @pl.when(jnp.logical_and(nxt < n, nxt % 2 == 0))
    def _():
        _copy(nxt, buf0, sem0).start()

    @pl.when(jnp.logical_and(nxt < n, nxt % 2 == 1))
    def _():
        _copy(nxt, buf1, sem1).start()

    @pl.when(i == 0)
    def _():
        loss_ref[0, 0] = 0.0

    def _do(buf, sem):
        _copy(i, buf, sem).wait()
        zqt = buf[...].T
        zqt_ref[...] = zqt
        diff = zqt - ze_ref[...]
        loss_ref[0, 0] += jnp.sum(diff * diff)

    @pl.when(i % 2 == 0)
    def _():
        _do(buf0, sem0)

    @pl.when(i % 2 == 1)
    def _():
        _do(buf1, sem1)


def _transpose_and_loss(z_q_raw, z_e_r):
    grid = (_N_Z // _T_BLK,)
    return pl.pallas_call(
        _loss_transpose_body,
        grid=grid,
        in_specs=[
            pl.BlockSpec(memory_space=pl.ANY),
            pl.BlockSpec((_E_DIM, _T_BLK), lambda i: (0, i)),
        ],
        out_specs=[
            pl.BlockSpec((_E_DIM, _T_BLK), lambda i: (0, i)),
            pl.BlockSpec(memory_space=pltpu.SMEM),
        ],
        out_shape=[
            jax.ShapeDtypeStruct((_E_DIM, _N_Z), jnp.float32),
            jax.ShapeDtypeStruct((1, 1), jnp.float32),
        ],
        scratch_shapes=[
            pltpu.VMEM((_T_BLK, _E_DIM), jnp.float32),
            pltpu.VMEM((_T_BLK, _E_DIM), jnp.float32),
            pltpu.SemaphoreType.DMA,
            pltpu.SemaphoreType.DMA,
        ],
    )(z_q_raw, z_e_r)


def kernel(z_e, embedding_weight):
    z_shape = z_e.shape
    flatten = jnp.transpose(z_e, (0, 2, 3, 1)).reshape(-1, _E_DIM)
    zn = jnp.sum(flatten**2, axis=1)
    en = jnp.sum(embedding_weight**2, axis=1)
    idx = _compute_indices(
        flatten, embedding_weight, zn.reshape(-1, 1), en.reshape(1, -1)
    )
    z_q_raw = _gather_rows(embedding_weight, idx.reshape(-1))
    z_e_r = z_e.reshape(_E_DIM, _N_Z)
    z_q_t, loss_sum = _transpose_and_loss(z_q_raw, z_e_r)
    z_q = z_q_t.reshape(z_shape)
    t = loss_sum[0, 0] / jnp.float32(z_e.size)
    vq_loss = t + _COMMITMENT_COST * t
    return (z_q, vq_loss)


# speedup vs baseline: 1.3736x; 1.0048x over previous
import functools

import jax
import jax.numpy as jnp
from jax import lax
from jax.experimental import pallas as pl
from jax.experimental.pallas import tpu as pltpu
from jax.experimental.pallas import tpu_sc as plsc

_N_E = 8192
_E_DIM = 256
_N_Z = 8192
_COMMITMENT_COST = 0.25

_R_BLK = 1024
_C_BLK = 2048

_T_BLK = 1024


def _argmin_body(x_ref, w_ref, zn_ref, en_ref, idx_ref, minval_ref, minidx_ref):
    j = pl.program_id(1)

    @pl.when(j == 0)
    def _():
        minval_ref[...] = jnp.full_like(minval_ref[...], jnp.inf)
        minidx_ref[...] = jnp.zeros_like(minidx_ref[...])

    xb = (x_ref[...] * -2.0).astype(jnp.bfloat16)
    wb = w_ref[...].astype(jnp.bfloat16)
    mm2 = lax.dot_general(
        xb, wb, (((1,), (1,)), ((), ())), preferred_element_type=jnp.float32
    )
    zn = zn_ref[...]
    en = en_ref[...]
    mv = minval_ref[...]
    iv = minidx_ref[...]
    lane = lax.broadcasted_iota(jnp.int32, (1, 128), 1)
    for c in range(_C_BLK // 128):
        sl = slice(c * 128, (c + 1) * 128)
        d_c = (zn + en[:, sl]) + mm2[:, sl]
        colidx = lane + (j * _C_BLK + c * 128)
        better = d_c < mv
        mv = jnp.where(better, d_c, mv)
        iv = jnp.where(better, colidx, iv)
    minval_ref[...] = mv
    minidx_ref[...] = iv

    @pl.when(j == pl.num_programs(1) - 1)
    def _():
        m = jnp.min(mv, axis=1, keepdims=True)
        li = jnp.min(
            jnp.where(mv == m, iv, jnp.int32(2**30)), axis=1, keepdims=True
        )
        idx_ref[...] = li


def _compute_indices(flatten, w, zn, en):
    grid = (_N_Z // _R_BLK, _N_E // _C_BLK)
    return pl.pallas_call(
        _argmin_body,
        grid=grid,
        in_specs=[
            pl.BlockSpec((_R_BLK, _E_DIM), lambda i, j: (i, 0)),
            pl.BlockSpec((_C_BLK, _E_DIM), lambda i, j: (j, 0)),
            pl.BlockSpec((_R_BLK, 1), lambda i, j: (i, 0)),
            pl.BlockSpec((1, _C_BLK), lambda i, j: (0, j)),
        ],
        out_specs=pl.BlockSpec((_R_BLK, 1), lambda i, j: (i, 0)),
        out_shape=jax.ShapeDtypeStruct((_N_Z, 1), jnp.int32),
        scratch_shapes=[
            pltpu.VMEM((_R_BLK, 128), jnp.float32),
            pltpu.VMEM((_R_BLK, 128), jnp.int32),
        ],
    )(flatten, w, zn, en)


_SC_MESH = plsc.VectorSubcoreMesh(core_axis_name="c", subcore_axis_name="s")
_NUM_WORKERS = 32
_B_PER_W = _N_Z // _NUM_WORKERS


def _gather_rows(w, idx):
    @functools.partial(
        pl.kernel,
        mesh=_SC_MESH,
        out_type=jax.ShapeDtypeStruct((_N_Z, _E_DIM), jnp.float32),
        scratch_types=[
            pltpu.VMEM((_B_PER_W,), jnp.int32),
            pltpu.VMEM((_B_PER_W, _E_DIM), jnp.float32),
            pltpu.SemaphoreType.DMA,
        ],
    )
    def k(table_hbm, idx_hbm, out_hbm, idx_v, rows_v, sem):
        wid = lax.axis_index("s") * 2 + lax.axis_index("c")
        base = wid * _B_PER_W
        pltpu.sync_copy(idx_hbm.at[pl.ds(base, _B_PER_W)], idx_v)
        pltpu.async_copy(table_hbm.at[idx_v], rows_v, sem).wait()
        pltpu.sync_copy(rows_v, out_hbm.at[pl.ds(base, _B_PER_W)])

    return k(w, idx)


def _loss_transpose_body(zq_ref, ze_ref, zqt_ref, loss_ref):
    i = pl.program_id(0)
    zqt = zq_ref[...].T
    zqt_ref[...] = zqt
    diff = zqt - ze_ref[...]

    @pl.when(i == 0)
    def _():
        loss_ref[0, 0] = 0.0

    loss_ref[0, 0] += jnp.sum(diff * diff)


def _transpose_and_loss(z_q_raw, z_e_r):
    grid = (_N_Z // _T_BLK,)
    return pl.pallas_call(
        _loss_transpose_body,
        grid=grid,
        in_specs=[
            pl.BlockSpec((_T_BLK, _E_DIM), lambda i: (i, 0)),
            pl.BlockSpec((_E_DIM, _T_BLK), lambda i: (0, i)),
        ],
        out_specs=[
            pl.BlockSpec((_E_DIM, _T_BLK), lambda i: (0, i)),
            pl.BlockSpec(memory_space=pltpu.SMEM),
        ],
        out_shape=[
            jax.ShapeDtypeStruct((_E_DIM, _N_Z), jnp.float32),
            jax.ShapeDtypeStruct((1, 1), jnp.float32),
        ],
    )(z_q_raw, z_e_r)


def kernel(z_e, embedding_weight):
    z_shape = z_e.shape
    flatten = jnp.transpose(z_e, (0, 2, 3, 1)).reshape(-1, _E_DIM)
    zn = jnp.sum(flatten**2, axis=1)
    en = jnp.sum(embedding_weight**2, axis=1)
    idx = _compute_indices(
        flatten, embedding_weight, zn.reshape(-1, 1), en.reshape(1, -1)
    )
    z_q_raw = _gather_rows(embedding_weight, idx.reshape(-1))
    z_e_r = z_e.reshape(_E_DIM, _N_Z)
    z_q_t, loss_sum = _transpose_and_loss(z_q_raw, z_e_r)
    z_q = z_q_t.reshape(z_shape)
    t = loss_sum[0, 0] / jnp.float32(z_e.size)
    vq_loss = t + _COMMITMENT_COST * t
    return (z_q, vq_loss)


# speedup vs baseline: 1.4184x; 1.0326x over previous
import functools

import jax
import jax.numpy as jnp
from jax import lax
from jax.experimental import pallas as pl
from jax.experimental.pallas import tpu as pltpu
from jax.experimental.pallas import tpu_sc as plsc

_N_E = 8192
_E_DIM = 256
_N_Z = 8192
_COMMITMENT_COST = 0.25

_R_BLK = 4096
_C_BLK = 1024

_T_BLK = 1024


def _argmin_body(x_ref, w_ref, zn_ref, en_ref, idx_ref, minval_ref, minidx_ref):
    j = pl.program_id(1)

    @pl.when(j == 0)
    def _():
        minval_ref[...] = jnp.full_like(minval_ref[...], jnp.inf)
        minidx_ref[...] = jnp.zeros_like(minidx_ref[...])

    xb = (x_ref[...] * -2.0).astype(jnp.bfloat16)
    wb = w_ref[...].astype(jnp.bfloat16)
    mm2 = lax.dot_general(
        xb, wb, (((1,), (1,)), ((), ())), preferred_element_type=jnp.float32
    )
    zn = zn_ref[...]
    en = en_ref[...]
    mv = minval_ref[...]
    iv = minidx_ref[...]
    lane = lax.broadcasted_iota(jnp.int32, (1, 128), 1)
    for c in range(_C_BLK // 128):
        sl = slice(c * 128, (c + 1) * 128)
        d_c = (zn + en[:, sl]) + mm2[:, sl]
        colidx = lane + (j * _C_BLK + c * 128)
        better = d_c < mv
        mv = jnp.where(better, d_c, mv)
        iv = jnp.where(better, colidx, iv)
    minval_ref[...] = mv
    minidx_ref[...] = iv

    @pl.when(j == pl.num_programs(1) - 1)
    def _():
        m = jnp.min(mv, axis=1, keepdims=True)
        li = jnp.min(
            jnp.where(mv == m, iv, jnp.int32(2**30)), axis=1, keepdims=True
        )
        idx_ref[...] = li


def _compute_indices(flatten, w, zn, en):
    grid = (_N_Z // _R_BLK, _N_E // _C_BLK)
    return pl.pallas_call(
        _argmin_body,
        grid=grid,
        in_specs=[
            pl.BlockSpec((_R_BLK, _E_DIM), lambda i, j: (i, 0)),
            pl.BlockSpec((_C_BLK, _E_DIM), lambda i, j: (j, 0)),
            pl.BlockSpec((_R_BLK, 1), lambda i, j: (i, 0)),
            pl.BlockSpec((1, _C_BLK), lambda i, j: (0, j)),
        ],
        out_specs=pl.BlockSpec((_R_BLK, 1), lambda i, j: (i, 0)),
        out_shape=jax.ShapeDtypeStruct((_N_Z, 1), jnp.int32),
        scratch_shapes=[
            pltpu.VMEM((_R_BLK, 128), jnp.float32),
            pltpu.VMEM((_R_BLK, 128), jnp.int32),
        ],
    )(flatten, w, zn, en)


_SC_MESH = plsc.VectorSubcoreMesh(core_axis_name="c", subcore_axis_name="s")
_NUM_WORKERS = 32
_B_PER_W = _N_Z // _NUM_WORKERS


def _gather_rows(w, idx):
    @functools.partial(
        pl.kernel,
        mesh=_SC_MESH,
        out_type=jax.ShapeDtypeStruct((_N_Z, _E_DIM), jnp.float32),
        scratch_types=[
            pltpu.VMEM((_B_PER_W,), jnp.int32),
            pltpu.VMEM((_B_PER_W, _E_DIM), jnp.float32),
            pltpu.SemaphoreType.DMA,
        ],
    )
    def k(table_hbm, idx_hbm, out_hbm, idx_v, rows_v, sem):
        wid = lax.axis_index("s") * 2 + lax.axis_index("c")
        base = wid * _B_PER_W
        pltpu.sync_copy(idx_hbm.at[pl.ds(base, _B_PER_W)], idx_v)
        pltpu.async_copy(table_hbm.at[idx_v], rows_v, sem).wait()
        pltpu.sync_copy(rows_v, out_hbm.at[pl.ds(base, _B_PER_W)])

    return k(w, idx)


def _loss_transpose_body(zq_ref, ze_ref, zqt_ref, loss_ref):
    i = pl.program_id(0)
    zqt = zq_ref[...].T
    zqt_ref[...] = zqt
    diff = zqt - ze_ref[...]

    @pl.when(i == 0)
    def _():
        loss_ref[0, 0] = 0.0

    loss_ref[0, 0] += jnp.sum(diff * diff)


def _transpose_and_loss(z_q_raw, z_e_r):
    grid = (_N_Z // _T_BLK,)
    return pl.pallas_call(
        _loss_transpose_body,
        grid=grid,
        in_specs=[
            pl.BlockSpec((_T_BLK, _E_DIM), lambda i: (i, 0)),
            pl.BlockSpec((_E_DIM, _T_BLK), lambda i: (0, i)),
        ],
        out_specs=[
            pl.BlockSpec((_E_DIM, _T_BLK), lambda i: (0, i)),
            pl.BlockSpec(memory_space=pltpu.SMEM),
        ],
        out_shape=[
            jax.ShapeDtypeStruct((_E_DIM, _N_Z), jnp.float32),
            jax.ShapeDtypeStruct((1, 1), jnp.float32),
        ],
    )(z_q_raw, z_e_r)


def kernel(z_e, embedding_weight):
    z_shape = z_e.shape
    flatten = jnp.transpose(z_e, (0, 2, 3, 1)).reshape(-1, _E_DIM)
    zn = jnp.sum(flatten**2, axis=1)
    en = jnp.sum(embedding_weight**2, axis=1)
    idx = _compute_indices(
        flatten, embedding_weight, zn.reshape(-1, 1), en.reshape(1, -1)
    )
    z_q_raw = _gather_rows(embedding_weight, idx.reshape(-1))
    z_e_r = z_e.reshape(_E_DIM, _N_Z)
    z_q_t, loss_sum = _transpose_and_loss(z_q_raw, z_e_r)
    z_q = z_q_t.reshape(z_shape)
    t = loss_sum[0, 0] / jnp.float32(z_e.size)
    vq_loss = t + _COMMITMENT_COST * t
    return (z_q, vq_loss)


# speedup vs baseline: 1.4288x; 1.0073x over previous
import functools

import jax
import jax.numpy as jnp
from jax import lax
from jax.experimental import pallas as pl
from jax.experimental.pallas import tpu as pltpu
from jax.experimental.pallas import tpu_sc as plsc

_N_E = 8192
_E_DIM = 256
_N_Z = 8192
_COMMITMENT_COST = 0.25

_R_BLK = 8192
_C_BLK = 1024

_T_BLK = 1024


def _argmin_body(x_ref, w_ref, zn_ref, en_ref, idx_ref, minval_ref, minidx_ref):
    j = pl.program_id(1)

    @pl.when(j == 0)
    def _():
        minval_ref[...] = jnp.full_like(minval_ref[...], jnp.inf)
        minidx_ref[...] = jnp.zeros_like(minidx_ref[...])

    xb = (x_ref[...] * -2.0).astype(jnp.bfloat16)
    wb = w_ref[...].astype(jnp.bfloat16)
    mm2 = lax.dot_general(
        xb, wb, (((1,), (1,)), ((), ())), preferred_element_type=jnp.float32
    )
    zn = zn_ref[...]
    en = en_ref[...]
    mv = minval_ref[...]
    iv = minidx_ref[...]
    lane = lax.broadcasted_iota(jnp.int32, (1, 128), 1)
    for c in range(_C_BLK // 128):
        sl = slice(c * 128, (c + 1) * 128)
        d_c = (zn + en[:, sl]) + mm2[:, sl]
        colidx = lane + (j * _C_BLK + c * 128)
        better = d_c < mv
        mv = jnp.where(better, d_c, mv)
        iv = jnp.where(better, colidx, iv)
    minval_ref[...] = mv
    minidx_ref[...] = iv

    @pl.when(j == pl.num_programs(1) - 1)
    def _():
        m = jnp.min(mv, axis=1, keepdims=True)
        li = jnp.min(
            jnp.where(mv == m, iv, jnp.int32(2**30)), axis=1, keepdims=True
        )
        idx_ref[...] = li


def _compute_indices(flatten, w, zn, en):
    grid = (_N_Z // _R_BLK, _N_E // _C_BLK)
    return pl.pallas_call(
        _argmin_body,
        grid=grid,
        in_specs=[
            pl.BlockSpec((_R_BLK, _E_DIM), lambda i, j: (i, 0)),
            pl.BlockSpec((_C_BLK, _E_DIM), lambda i, j: (j, 0)),
            pl.BlockSpec((_R_BLK, 1), lambda i, j: (i, 0)),
            pl.BlockSpec((1, _C_BLK), lambda i, j: (0, j)),
        ],
        out_specs=pl.BlockSpec((_R_BLK, 1), lambda i, j: (i, 0)),
        out_shape=jax.ShapeDtypeStruct((_N_Z, 1), jnp.int32),
        scratch_shapes=[
            pltpu.VMEM((_R_BLK, 128), jnp.float32),
            pltpu.VMEM((_R_BLK, 128), jnp.int32),
        ],
    )(flatten, w, zn, en)


_SC_MESH = plsc.VectorSubcoreMesh(core_axis_name="c", subcore_axis_name="s")
_NUM_WORKERS = 32
_B_PER_W = _N_Z // _NUM_WORKERS


def _gather_rows(w, idx):
    @functools.partial(
        pl.kernel,
        mesh=_SC_MESH,
        out_type=jax.ShapeDtypeStruct((_N_Z, _E_DIM), jnp.float32),
        scratch_types=[
            pltpu.VMEM((_B_PER_W,), jnp.int32),
            pltpu.VMEM((_B_PER_W, _E_DIM), jnp.float32),
            pltpu.SemaphoreType.DMA,
        ],
    )
    def k(table_hbm, idx_hbm, out_hbm, idx_v, rows_v, sem):
        wid = lax.axis_index("s") * 2 + lax.axis_index("c")
        base = wid * _B_PER_W
        pltpu.sync_copy(idx_hbm.at[pl.ds(base, _B_PER_W)], idx_v)
        pltpu.async_copy(table_hbm.at[idx_v], rows_v, sem).wait()
        pltpu.sync_copy(rows_v, out_hbm.at[pl.ds(base, _B_PER_W)])

    return k(w, idx)


def _loss_transpose_body(zq_ref, ze_ref, zqt_ref, loss_ref):
    i = pl.program_id(0)
    zqt = zq_ref[...].T
    zqt_ref[...] = zqt
    diff = zqt - ze_ref[...]

    @pl.when(i == 0)
    def _():
        loss_ref[0, 0] = 0.0

    loss_ref[0, 0] += jnp.sum(diff * diff)


def _transpose_and_loss(z_q_raw, z_e_r):
    grid = (_N_Z // _T_BLK,)
    return pl.pallas_call(
        _loss_transpose_body,
        grid=grid,
        in_specs=[
            pl.BlockSpec((_T_BLK, _E_DIM), lambda i: (i, 0)),
            pl.BlockSpec((_E_DIM, _T_BLK), lambda i: (0, i)),
        ],
        out_specs=[
            pl.BlockSpec((_E_DIM, _T_BLK), lambda i: (0, i)),
            pl.BlockSpec(memory_space=pltpu.SMEM),
        ],
        out_shape=[
            jax.ShapeDtypeStruct((_E_DIM, _N_Z), jnp.float32),
            jax.ShapeDtypeStruct((1, 1), jnp.float32),
        ],
    )(z_q_raw, z_e_r)


def kernel(z_e, embedding_weight):
    z_shape = z_e.shape
    flatten = jnp.transpose(z_e, (0, 2, 3, 1)).reshape(-1, _E_DIM)
    zn = jnp.sum(flatten**2, axis=1)
    en = jnp.sum(embedding_weight**2, axis=1)
    idx = _compute_indices(
        flatten, embedding_weight, zn.reshape(-1, 1), en.reshape(1, -1)
    )
    z_q_raw = _gather_rows(embedding_weight, idx.reshape(-1))
    z_e_r = z_e.reshape(_E_DIM, _N_Z)
    z_q_t, loss_sum = _transpose_and_loss(z_q_raw, z_e_r)
    z_q = z_q_t.reshape(z_shape)
    t = loss_sum[0, 0] / jnp.float32(z_e.size)
    vq_loss = t + _COMMITMENT_COST * t
    return (z_q, vq_loss)
